# Initial kernel scaffold; baseline (speedup 1.0000x reference)
import jax
import jax.numpy as jnp
from jax.experimental import pallas as pl


def kernel(table, Wq, bq, Wk, bk, Wv, bv, node, neigh_ids):
    raise NotImplementedError("write your pallas kernel here")


# speedup vs baseline: 7.3016x; 7.3016x over previous
import jax
import jax.numpy as jnp
from jax import lax
from jax.experimental import pallas as pl
from jax.experimental.pallas import tpu as pltpu

_S = 64
_ROWS = _S + 1
_PAD = 72
_D = 128


def _body(node_ref, ids_ref, table_ref, wq, bq, wk, bk, wv, bv,
          out_ref, rows, sem):
    copies = []
    c = pltpu.make_async_copy(
        table_ref.at[pl.ds(node_ref[0], 1)], rows.at[pl.ds(0, 1)], sem)
    c.start()
    copies.append(c)
    for i in range(_S):
        c = pltpu.make_async_copy(
            table_ref.at[pl.ds(ids_ref[i], 1)], rows.at[pl.ds(1 + i, 1)], sem)
        c.start()
        copies.append(c)
    for c in copies:
        c.wait()

    row_id2 = lax.broadcasted_iota(jnp.int32, (_PAD, _D), 0)
    r = jnp.where(row_id2 < _ROWS, rows[...], 0.0)
    self_row = r[0:1]
    q = jnp.dot(self_row, wq[...],
                preferred_element_type=jnp.float32) + bq[...]
    k = jnp.dot(r, wk[...],
                preferred_element_type=jnp.float32) + bk[...]
    v = jnp.dot(r, wv[...],
                preferred_element_type=jnp.float32) + bv[...]

    s = jnp.dot(k, q.T, preferred_element_type=jnp.float32)
    row_id = lax.broadcasted_iota(jnp.int32, (_PAD, 1), 0)
    s = jnp.where(row_id < _ROWS, s, -jnp.inf)
    m = jnp.max(s)
    e = jnp.exp(s - m)
    p = e / jnp.sum(e)
    mix = jnp.sum(p * v, axis=0, keepdims=True)

    f = jnp.tanh(mix)
    norm = jnp.maximum(jnp.sqrt(jnp.sum(f * f)), 1e-12)
    out_ref[...] = f / norm


def kernel(table, Wq, bq, Wk, bk, Wv, bv, node, neigh_ids):
    node1 = jnp.reshape(node, (1,)).astype(jnp.int32)
    return pl.pallas_call(
        _body,
        out_shape=jax.ShapeDtypeStruct((1, _D), jnp.float32),
        in_specs=[
            pl.BlockSpec(memory_space=pltpu.SMEM),
            pl.BlockSpec(memory_space=pltpu.SMEM),
            pl.BlockSpec(memory_space=pl.ANY),
            pl.BlockSpec(memory_space=pltpu.VMEM),
            pl.BlockSpec(memory_space=pltpu.VMEM),
            pl.BlockSpec(memory_space=pltpu.VMEM),
            pl.BlockSpec(memory_space=pltpu.VMEM),
            pl.BlockSpec(memory_space=pltpu.VMEM),
            pl.BlockSpec(memory_space=pltpu.VMEM),
        ],
        out_specs=pl.BlockSpec(memory_space=pltpu.VMEM),
        scratch_shapes=[
            pltpu.VMEM((_PAD, _D), jnp.float32),
            pltpu.SemaphoreType.DMA,
        ],
    )(node1, neigh_ids, table,
      Wq, jnp.reshape(bq, (1, _D)),
      Wk, jnp.reshape(bk, (1, _D)),
      Wv, jnp.reshape(bv, (1, _D)))
